# baseline (speedup 1.0000x reference)
import jax
import jax.numpy as jnp
from jax.experimental import pallas as pl

USER_NUM = 10000
ITEM_NUM = 40000
NNODE = USER_NUM + ITEM_NUM


def _mlp_body(e_ref, w1_ref, b1_ref, w2_ref, b2_ref, w3_ref, b3_ref, o_ref):
    e = e_ref[...]
    h = jax.nn.relu(jnp.dot(e, w1_ref[...], preferred_element_type=jnp.float32) + b1_ref[...])
    h = jax.nn.relu(jnp.dot(h, w2_ref[...], preferred_element_type=jnp.float32) + b2_ref[...])
    o = jnp.dot(h, w3_ref[...], preferred_element_type=jnp.float32) + b3_ref[...]
    o_ref[...] = o


def kernel(userIdx, itemIdx, edge_src, edge_dst, edge_w, uEmbd, iEmbd,
           lin1_w, lin1_b, int1_w, int1_b, lin2_w, lin2_b, int2_w, int2_b,
           t1_w, t1_b, t2_w, t2_b, t3_w, t3_b):
    def spmm(feats):
        return jax.ops.segment_sum(edge_w[:, None] * feats[edge_src], edge_dst,
                                   num_segments=NNODE)

    features = jnp.concatenate([uEmbd, iEmbd], axis=0)
    f = features
    m1 = spmm(f) + f
    m2 = spmm(f * f)
    out = (m1 @ lin1_w.T + lin1_b) + (m2 @ int1_w.T + int1_b)
    f1 = jax.nn.relu(out)
    m1 = spmm(f1) + f1
    m2 = spmm(f1 * f1)
    out = (m1 @ lin2_w.T + lin2_b) + (m2 @ int2_w.T + int2_b)
    f2 = jax.nn.relu(out)
    final = jnp.concatenate([features, f1, f2], axis=1)
    itemIdx2 = itemIdx + USER_NUM
    uE = final[userIdx]
    iE = final[itemIdx2]
    e = jnp.concatenate([uE, iE], axis=1)

    B = e.shape[0]
    BLK = 512
    out = pl.pallas_call(
        _mlp_body,
        grid=(B // BLK,),
        in_specs=[
            pl.BlockSpec((BLK, 460), lambda i: (i, 0)),
            pl.BlockSpec((460, 64), lambda i: (0, 0)),
            pl.BlockSpec((64,), lambda i: (0,)),
            pl.BlockSpec((64, 32), lambda i: (0, 0)),
            pl.BlockSpec((32,), lambda i: (0,)),
            pl.BlockSpec((32, 1), lambda i: (0, 0)),
            pl.BlockSpec((1,), lambda i: (0,)),
        ],
        out_specs=pl.BlockSpec((BLK, 1), lambda i: (i, 0)),
        out_shape=jax.ShapeDtypeStruct((B, 1), jnp.float32),
    )(e, t1_w.T, t1_b, t2_w.T, t2_b, t3_w.T, t3_b)
    return out.reshape(-1)


# speedup vs baseline: 4.7901x; 4.7901x over previous
import functools
import jax
import jax.numpy as jnp
from jax import lax
from jax.experimental import pallas as pl
from jax.experimental.pallas import tpu as pltpu
from jax.experimental.pallas import tpu_sc as plsc

USER_NUM = 10000
ITEM_NUM = 40000
NNODE = USER_NUM + ITEM_NUM
NPAD = 50176
NEDGE = 800000
NCORE = 2
NSUB = 16
EPAD = 819200
EDGES_PER_SUB = EPAD // NSUB
TILE = 1024
G = TILE // 128
NTILE = EDGES_PER_SUB // TILE
ROWS_PER_SUB = NPAD // NSUB
BATCH = 4096

_f32 = jnp.float32


def _bcast16(vec, r):
    idx = jax.lax.broadcast(jnp.int32(r), (16,))
    return jax.lax.gather(
        vec, idx[:, None],
        jax.lax.GatherDimensionNumbers(
            offset_dims=(), collapsed_slice_dims=(0,), start_index_map=(0,)),
        (1,), mode=jax.lax.GatherScatterMode.PROMISE_IN_BOUNDS)


def _make_spmm(nchunk):
    cpc = (nchunk + 1) // 2
    mesh = plsc.VectorSubcoreMesh(core_axis_name="c", subcore_axis_name="s")

    @functools.partial(
        pl.kernel, mesh=mesh,
        compiler_params=pltpu.CompilerParams(use_tc_tiling_on_sc=False),
        out_type=jax.ShapeDtypeStruct((nchunk, NPAD, 16), _f32),
        scratch_types=[
            pltpu.VMEM((G, 128), jnp.int32),
            pltpu.VMEM((G, 128), jnp.int32),
            pltpu.VMEM((G, 128), _f32),
            pltpu.VMEM((TILE, 16), _f32),
            pltpu.VMEM((ROWS_PER_SUB, 16), _f32),
            pltpu.VMEM_SHARED((NPAD, 16), _f32),
            pltpu.SemaphoreType.DMA,
        ],
    )
    def spmm(z_hbm, src_hbm, dst_hbm, w_hbm, out_hbm,
             src_v, dst_v, w_v, rows_v, zbuf, acc_sh, sem):
        c = lax.axis_index("c")
        s = lax.axis_index("s")

        def zb(i, x):
            zbuf[i] = jnp.zeros((16,), _f32)
            return x
        lax.fori_loop(0, ROWS_PER_SUB, zb, 0)
        row0 = s * ROWS_PER_SUB

        for t in range(cpc):
            j = t * 2 + c

            @pl.when(j < nchunk)
            def _chunk():
                pltpu.sync_copy(zbuf, acc_sh.at[pl.ds(row0, ROWS_PER_SUB)])
                plsc.subcore_barrier()

                def tile_body(ti, x):
                    roff = s * (EDGES_PER_SUB // 128) + ti * G
                    pltpu.sync_copy(src_hbm.at[pl.ds(roff, G)], src_v)
                    pltpu.sync_copy(dst_hbm.at[pl.ds(roff, G)], dst_v)
                    pltpu.sync_copy(w_hbm.at[pl.ds(roff, G)], w_v)
                    for g in range(G):
                        pltpu.async_copy(
                            z_hbm.at[j].at[src_v.at[g]],
                            rows_v.at[pl.ds(g * 128, 128)], sem).wait()
                    for g in range(G):
                        def mulbody(e16, y, g=g):
                            wv = w_v[g, pl.ds(e16 * 16, 16)]
                            base = g * 128 + e16 * 16
                            for r in range(16):
                                rows_v[base + r] = rows_v[base + r] * _bcast16(wv, r)
                            return y
                        lax.fori_loop(0, 8, mulbody, 0)
                    for g in range(G):
                        pltpu.sync_copy(
                            rows_v.at[pl.ds(g * 128, 128)],
                            acc_sh.at[dst_v.at[g]], add=True)
                    return x
                lax.fori_loop(0, NTILE, tile_body, 0)
                plsc.subcore_barrier()
                pltpu.sync_copy(acc_sh.at[pl.ds(row0, ROWS_PER_SUB)],
                                out_hbm.at[j, pl.ds(row0, ROWS_PER_SUB)])
    return spmm


_spmm5 = _make_spmm(5)
_spmm4 = _make_spmm(4)


_BLK = 1024


def _ka_body(f_ref, w1_ref, wi_ref, b_ref, z_ref, pb_ref):
    f = f_ref[...]
    p = jnp.dot(f, w1_ref[...], preferred_element_type=_f32)
    q = jnp.dot(f * f, wi_ref[...], preferred_element_type=_f32)
    z = p + q
    z_ref[...] = jnp.transpose(z.reshape(_BLK, 5, 16), (1, 0, 2))
    pb_ref[...] = p + b_ref[...]


def _kc_body(s_ref, pb_ref, w2_ref, wi2_ref, b2_ref, z2_ref, p2b_ref, f1_ref):
    sflat = jnp.transpose(s_ref[...], (1, 0, 2)).reshape(_BLK, 80)
    f1 = jax.nn.relu(sflat + pb_ref[...])
    f1_ref[...] = f1
    p = jnp.dot(f1, w2_ref[...], preferred_element_type=_f32)
    q = jnp.dot(f1 * f1, wi2_ref[...], preferred_element_type=_f32)
    z = p + q
    z2_ref[...] = jnp.transpose(z.reshape(_BLK, 4, 16), (1, 0, 2))
    p2b_ref[...] = p + b2_ref[...]


def _ke_body(s_ref, p2b_ref, f2_ref):
    sflat = jnp.transpose(s_ref[...], (1, 0, 2)).reshape(_BLK, 64)
    f2_ref[...] = jax.nn.relu(sflat + p2b_ref[...])


def _make_gatherk():
    mesh = plsc.VectorSubcoreMesh(core_axis_name="c", subcore_axis_name="s")
    nw = NCORE * NSUB
    bpw = 2 * BATCH // nw

    @functools.partial(
        pl.kernel, mesh=mesh,
        compiler_params=pltpu.CompilerParams(use_tc_tiling_on_sc=False),
        out_type=[jax.ShapeDtypeStruct((2 * BATCH, 112), _f32),
                  jax.ShapeDtypeStruct((2 * BATCH, 80), _f32),
                  jax.ShapeDtypeStruct((2 * BATCH, 64), _f32)],
        scratch_types=[
            pltpu.VMEM((2, 128), jnp.int32),
            pltpu.VMEM((bpw, 112), _f32),
            pltpu.VMEM((bpw, 80), _f32),
            pltpu.VMEM((bpw, 64), _f32),
            pltpu.SemaphoreType.DMA,
        ],
    )
    def gk(fp_hbm, f1_hbm, f2_hbm, idx_hbm, gf_hbm, g1_hbm, g2_hbm,
           idx_v, bf, b1, b2, sem):
        c = lax.axis_index("c")
        s = lax.axis_index("s")
        wid = s * NCORE + c
        base = wid * bpw
        pltpu.sync_copy(idx_hbm.at[wid], idx_v)
        for g in range(2):
            pltpu.async_copy(fp_hbm.at[idx_v.at[g]],
                             bf.at[pl.ds(g * 128, 128)], sem).wait()
            pltpu.async_copy(f1_hbm.at[idx_v.at[g]],
                             b1.at[pl.ds(g * 128, 128)], sem).wait()
            pltpu.async_copy(f2_hbm.at[idx_v.at[g]],
                             b2.at[pl.ds(g * 128, 128)], sem).wait()
        pltpu.sync_copy(bf, gf_hbm.at[pl.ds(base, bpw)])
        pltpu.sync_copy(b1, g1_hbm.at[pl.ds(base, bpw)])
        pltpu.sync_copy(b2, g2_hbm.at[pl.ds(base, bpw)])
    return gk


_gatherk = _make_gatherk()

_BLKB = 512


def _kg_body(gfu, gfi, g1u, g1i, g2u, g2i,
             wfu, wfi, w1u, w1i, w2u, w2i, b1, w2, b2, w3, b3, o_ref):
    h = (jnp.dot(gfu[0], wfu[...], preferred_element_type=_f32)
         + jnp.dot(gfi[0], wfi[...], preferred_element_type=_f32)
         + jnp.dot(g1u[0], w1u[...], preferred_element_type=_f32)
         + jnp.dot(g1i[0], w1i[...], preferred_element_type=_f32)
         + jnp.dot(g2u[0], w2u[...], preferred_element_type=_f32)
         + jnp.dot(g2i[0], w2i[...], preferred_element_type=_f32)) + b1[...]
    h = jax.nn.relu(h)
    h = jax.nn.relu(jnp.dot(h, w2[...], preferred_element_type=_f32) + b2[...])
    o_ref[...] = jnp.dot(h, w3[...], preferred_element_type=_f32) + b3[...]


def kernel(userIdx, itemIdx, edge_src, edge_dst, edge_w, uEmbd, iEmbd,
           lin1_w, lin1_b, int1_w, int1_b, lin2_w, lin2_b, int2_w, int2_b,
           t1_w, t1_b, t2_w, t2_b, t3_w, t3_b):
    features = jnp.concatenate([uEmbd, iEmbd], axis=0)

    pad = EPAD - NEDGE
    padidx = (jnp.arange(pad, dtype=jnp.int32) * 61) % NNODE
    src_p = jnp.concatenate([edge_src, padidx]).reshape(EPAD // 128, 128)
    dst_p = jnp.concatenate([edge_dst, padidx]).reshape(EPAD // 128, 128)
    w_p = jnp.concatenate([edge_w, jnp.zeros((pad,), _f32)]).reshape(EPAD // 128, 128)

    f_in = jnp.pad(features, ((0, NPAD - NNODE), (0, 0)))

    nblk = NPAD // _BLK
    z1, p1b = pl.pallas_call(
        _ka_body,
        grid=(nblk,),
        in_specs=[
            pl.BlockSpec((_BLK, 100), lambda i: (i, 0)),
            pl.BlockSpec((100, 80), lambda i: (0, 0)),
            pl.BlockSpec((100, 80), lambda i: (0, 0)),
            pl.BlockSpec((80,), lambda i: (0,)),
        ],
        out_specs=[
            pl.BlockSpec((5, _BLK, 16), lambda i: (0, i, 0)),
            pl.BlockSpec((_BLK, 80), lambda i: (i, 0)),
        ],
        out_shape=[
            jax.ShapeDtypeStruct((5, NPAD, 16), _f32),
            jax.ShapeDtypeStruct((NPAD, 80), _f32),
        ],
    )(f_in, lin1_w.T, int1_w.T, lin1_b + int1_b)

    s1 = _spmm5(z1, src_p, dst_p, w_p)

    w2T = jnp.pad(lin2_w.T, ((0, 0), (0, 14)))
    wi2T = jnp.pad(int2_w.T, ((0, 0), (0, 14)))
    b2 = jnp.pad(lin2_b + int2_b, (0, 14))
    z2, p2b, f1 = pl.pallas_call(
        _kc_body,
        grid=(nblk,),
        in_specs=[
            pl.BlockSpec((5, _BLK, 16), lambda i: (0, i, 0)),
            pl.BlockSpec((_BLK, 80), lambda i: (i, 0)),
            pl.BlockSpec((80, 64), lambda i: (0, 0)),
            pl.BlockSpec((80, 64), lambda i: (0, 0)),
            pl.BlockSpec((64,), lambda i: (0,)),
        ],
        out_specs=[
            pl.BlockSpec((4, _BLK, 16), lambda i: (0, i, 0)),
            pl.BlockSpec((_BLK, 64), lambda i: (i, 0)),
            pl.BlockSpec((_BLK, 80), lambda i: (i, 0)),
        ],
        out_shape=[
            jax.ShapeDtypeStruct((4, NPAD, 16), _f32),
            jax.ShapeDtypeStruct((NPAD, 64), _f32),
            jax.ShapeDtypeStruct((NPAD, 80), _f32),
        ],
    )(s1, p1b, w2T, wi2T, b2)

    s2 = _spmm4(z2, src_p, dst_p, w_p)

    f2 = pl.pallas_call(
        _ke_body,
        grid=(nblk,),
        in_specs=[
            pl.BlockSpec((4, _BLK, 16), lambda i: (0, i, 0)),
            pl.BlockSpec((_BLK, 64), lambda i: (i, 0)),
        ],
        out_specs=pl.BlockSpec((_BLK, 64), lambda i: (i, 0)),
        out_shape=jax.ShapeDtypeStruct((NPAD, 64), _f32),
    )(s2, p2b)

    features_p = jnp.pad(features, ((0, NPAD - NNODE), (0, 12)))
    cat_idx = jnp.concatenate([userIdx, itemIdx + USER_NUM]).reshape(
        NCORE * NSUB, 2, 128)
    gf, g1, g2 = _gatherk(features_p, f1, f2, cat_idx)

    W = t1_w.T
    wfu = jnp.pad(W[0:100], ((0, 12), (0, 0)))
    w1u = W[100:180]
    w2u = jnp.pad(W[180:230], ((0, 14), (0, 0)))
    wfi = jnp.pad(W[230:330], ((0, 12), (0, 0)))
    w1i = W[330:410]
    w2i = jnp.pad(W[410:460], ((0, 14), (0, 0)))
    gf3 = gf.reshape(2, BATCH, 112)
    g13 = g1.reshape(2, BATCH, 80)
    g23 = g2.reshape(2, BATCH, 64)

    out = pl.pallas_call(
        _kg_body,
        grid=(BATCH // _BLKB,),
        in_specs=[
            pl.BlockSpec((1, _BLKB, 112), lambda i: (0, i, 0)),
            pl.BlockSpec((1, _BLKB, 112), lambda i: (1, i, 0)),
            pl.BlockSpec((1, _BLKB, 80), lambda i: (0, i, 0)),
            pl.BlockSpec((1, _BLKB, 80), lambda i: (1, i, 0)),
            pl.BlockSpec((1, _BLKB, 64), lambda i: (0, i, 0)),
            pl.BlockSpec((1, _BLKB, 64), lambda i: (1, i, 0)),
            pl.BlockSpec((112, 64), lambda i: (0, 0)),
            pl.BlockSpec((112, 64), lambda i: (0, 0)),
            pl.BlockSpec((80, 64), lambda i: (0, 0)),
            pl.BlockSpec((80, 64), lambda i: (0, 0)),
            pl.BlockSpec((64, 64), lambda i: (0, 0)),
            pl.BlockSpec((64, 64), lambda i: (0, 0)),
            pl.BlockSpec((64,), lambda i: (0,)),
            pl.BlockSpec((64, 32), lambda i: (0, 0)),
            pl.BlockSpec((32,), lambda i: (0,)),
            pl.BlockSpec((32, 1), lambda i: (0, 0)),
            pl.BlockSpec((1,), lambda i: (0,)),
        ],
        out_specs=pl.BlockSpec((_BLKB, 1), lambda i: (i, 0)),
        out_shape=jax.ShapeDtypeStruct((BATCH, 1), _f32),
    )(gf3, gf3, g13, g13, g23, g23,
      wfu, wfi, w1u, w1i, w2u, w2i, t1_b, t2_w.T, t2_b, t3_w.T, t3_b)
    return out.reshape(-1)


# speedup vs baseline: 10.8266x; 2.2602x over previous
import functools
import jax
import jax.numpy as jnp
from jax import lax
from jax.experimental import pallas as pl
from jax.experimental.pallas import tpu as pltpu
from jax.experimental.pallas import tpu_sc as plsc

USER_NUM = 10000
ITEM_NUM = 40000
NNODE = USER_NUM + ITEM_NUM
NPAD = 50176
NEDGE = 800000
NCORE = 2
NSUB = 16
EPAD = 819200
EDGES_PER_SUB = EPAD // NSUB
TILE = 2048
G = TILE // 128
NTILE = EDGES_PER_SUB // TILE
ROWS_PER_SUB = NPAD // NSUB
BATCH = 4096

_f32 = jnp.float32


def _bcast16(vec, r):
    idx = jax.lax.broadcast(jnp.int32(r), (16,))
    return jax.lax.gather(
        vec, idx[:, None],
        jax.lax.GatherDimensionNumbers(
            offset_dims=(), collapsed_slice_dims=(0,), start_index_map=(0,)),
        (1,), mode=jax.lax.GatherScatterMode.PROMISE_IN_BOUNDS)


def _make_spmm(nchunk):
    cpc = (nchunk + 1) // 2
    mesh = plsc.VectorSubcoreMesh(core_axis_name="c", subcore_axis_name="s")

    @functools.partial(
        pl.kernel, mesh=mesh,
        compiler_params=pltpu.CompilerParams(use_tc_tiling_on_sc=False),
        out_type=jax.ShapeDtypeStruct((nchunk, NPAD, 16), _f32),
        scratch_types=[
            pltpu.VMEM((G, 128), jnp.int32),
            pltpu.VMEM((G, 128), jnp.int32),
            pltpu.VMEM((G, 128), _f32),
            pltpu.VMEM((TILE, 16), _f32),
            pltpu.VMEM_SHARED((NPAD, 16), _f32),
            pltpu.SemaphoreType.DMA,
            pltpu.SemaphoreType.DMA,
            pltpu.SemaphoreType.DMA,
        ],
    )
    def spmm(z_hbm, src_hbm, dst_hbm, w_hbm, out_hbm,
             src_v, dst_v, w_v, rows_v, acc_sh, sem, semi, sems):
        c = lax.axis_index("c")
        s = lax.axis_index("s")
        row0 = s * ROWS_PER_SUB

        for t in range(cpc):
            j = t * 2 + c

            @pl.when(j < nchunk)
            def _chunk():
                def zb(i, x):
                    rows_v[i] = jnp.zeros((16,), _f32)
                    return x
                lax.fori_loop(0, TILE, zb, 0)
                pltpu.sync_copy(rows_v, acc_sh.at[pl.ds(row0, TILE)])
                pltpu.sync_copy(rows_v.at[pl.ds(0, ROWS_PER_SUB - TILE)],
                                acc_sh.at[pl.ds(row0 + TILE, ROWS_PER_SUB - TILE)])
                plsc.subcore_barrier()

                def tile_body(ti, x):
                    roff = s * (EDGES_PER_SUB // 128) + ti * G
                    d1 = pltpu.async_copy(src_hbm.at[pl.ds(roff, G)], src_v, semi)
                    d2 = pltpu.async_copy(dst_hbm.at[pl.ds(roff, G)], dst_v, semi)
                    d3 = pltpu.async_copy(w_hbm.at[pl.ds(roff, G)], w_v, semi)
                    d1.wait(); d2.wait(); d3.wait()
                    gds = [pltpu.async_copy(
                               z_hbm.at[j].at[src_v.at[g]],
                               rows_v.at[pl.ds(g * 128, 128)], sem)
                           for g in range(G)]
                    for g in range(G):
                        gds[g].wait()
                        def mulbody(e16, y, g=g):
                            wv = w_v[g, pl.ds(e16 * 16, 16)]
                            base = g * 128 + e16 * 16
                            for r in range(16):
                                rows_v[base + r] = rows_v[base + r] * _bcast16(wv, r)
                            return y
                        lax.fori_loop(0, 8, mulbody, 0)
                        pltpu.async_copy(rows_v.at[pl.ds(g * 128, 128)],
                                         acc_sh.at[dst_v.at[g]], sems, add=True)
                    for g in range(G):
                        pltpu.make_async_copy(rows_v.at[pl.ds(g * 128, 128)],
                                              acc_sh.at[dst_v.at[g]], sems).wait()
                    return x
                lax.fori_loop(0, NTILE, tile_body, 0)
                plsc.subcore_barrier()
                pltpu.sync_copy(acc_sh.at[pl.ds(row0, ROWS_PER_SUB)],
                                out_hbm.at[j, pl.ds(row0, ROWS_PER_SUB)])
    return spmm


_spmm5 = _make_spmm(5)
_spmm4 = _make_spmm(4)


_BLK = 1024


def _ka_body(f_ref, w1_ref, wi_ref, b_ref, z_ref, pb_ref):
    f = f_ref[...]
    p = jnp.dot(f, w1_ref[...], preferred_element_type=_f32)
    q = jnp.dot(f * f, wi_ref[...], preferred_element_type=_f32)
    z = p + q
    z_ref[...] = jnp.transpose(z.reshape(_BLK, 5, 16), (1, 0, 2))
    pb_ref[...] = p + b_ref[...]


def _kc_body(s_ref, pb_ref, w2_ref, wi2_ref, b2_ref, z2_ref, p2b_ref, f1_ref):
    sflat = jnp.transpose(s_ref[...], (1, 0, 2)).reshape(_BLK, 80)
    f1 = jax.nn.relu(sflat + pb_ref[...])
    f1_ref[...] = f1
    p = jnp.dot(f1, w2_ref[...], preferred_element_type=_f32)
    q = jnp.dot(f1 * f1, wi2_ref[...], preferred_element_type=_f32)
    z = p + q
    z2_ref[...] = jnp.transpose(z.reshape(_BLK, 4, 16), (1, 0, 2))
    p2b_ref[...] = p + b2_ref[...]


def _ke_body(s_ref, p2b_ref, f2_ref):
    sflat = jnp.transpose(s_ref[...], (1, 0, 2)).reshape(_BLK, 64)
    f2_ref[...] = jax.nn.relu(sflat + p2b_ref[...])


def _make_gatherk():
    mesh = plsc.VectorSubcoreMesh(core_axis_name="c", subcore_axis_name="s")
    nw = NCORE * NSUB
    bpw = 2 * BATCH // nw

    @functools.partial(
        pl.kernel, mesh=mesh,
        compiler_params=pltpu.CompilerParams(use_tc_tiling_on_sc=False),
        out_type=[jax.ShapeDtypeStruct((2 * BATCH, 112), _f32),
                  jax.ShapeDtypeStruct((2 * BATCH, 80), _f32),
                  jax.ShapeDtypeStruct((2 * BATCH, 64), _f32)],
        scratch_types=[
            pltpu.VMEM((2, 128), jnp.int32),
            pltpu.VMEM((bpw, 112), _f32),
            pltpu.VMEM((bpw, 80), _f32),
            pltpu.VMEM((bpw, 64), _f32),
            pltpu.SemaphoreType.DMA,
        ],
    )
    def gk(fp_hbm, f1_hbm, f2_hbm, idx_hbm, gf_hbm, g1_hbm, g2_hbm,
           idx_v, bf, b1, b2, sem):
        c = lax.axis_index("c")
        s = lax.axis_index("s")
        wid = s * NCORE + c
        base = wid * bpw
        pltpu.sync_copy(idx_hbm.at[wid], idx_v)
        for g in range(2):
            pltpu.async_copy(fp_hbm.at[idx_v.at[g]],
                             bf.at[pl.ds(g * 128, 128)], sem).wait()
            pltpu.async_copy(f1_hbm.at[idx_v.at[g]],
                             b1.at[pl.ds(g * 128, 128)], sem).wait()
            pltpu.async_copy(f2_hbm.at[idx_v.at[g]],
                             b2.at[pl.ds(g * 128, 128)], sem).wait()
        pltpu.sync_copy(bf, gf_hbm.at[pl.ds(base, bpw)])
        pltpu.sync_copy(b1, g1_hbm.at[pl.ds(base, bpw)])
        pltpu.sync_copy(b2, g2_hbm.at[pl.ds(base, bpw)])
    return gk


_gatherk = _make_gatherk()

_BLKB = 512


def _kg_body(gfu, gfi, g1u, g1i, g2u, g2i,
             wfu, wfi, w1u, w1i, w2u, w2i, b1, w2, b2, w3, b3, o_ref):
    h = (jnp.dot(gfu[0], wfu[...], preferred_element_type=_f32)
         + jnp.dot(gfi[0], wfi[...], preferred_element_type=_f32)
         + jnp.dot(g1u[0], w1u[...], preferred_element_type=_f32)
         + jnp.dot(g1i[0], w1i[...], preferred_element_type=_f32)
         + jnp.dot(g2u[0], w2u[...], preferred_element_type=_f32)
         + jnp.dot(g2i[0], w2i[...], preferred_element_type=_f32)) + b1[...]
    h = jax.nn.relu(h)
    h = jax.nn.relu(jnp.dot(h, w2[...], preferred_element_type=_f32) + b2[...])
    o_ref[...] = jnp.dot(h, w3[...], preferred_element_type=_f32) + b3[...]


def kernel(userIdx, itemIdx, edge_src, edge_dst, edge_w, uEmbd, iEmbd,
           lin1_w, lin1_b, int1_w, int1_b, lin2_w, lin2_b, int2_w, int2_b,
           t1_w, t1_b, t2_w, t2_b, t3_w, t3_b):
    features = jnp.concatenate([uEmbd, iEmbd], axis=0)

    pad = EPAD - NEDGE
    padidx = (jnp.arange(pad, dtype=jnp.int32) * 61) % NNODE
    src_p = jnp.concatenate([edge_src, padidx]).reshape(EPAD // 128, 128)
    dst_p = jnp.concatenate([edge_dst, padidx]).reshape(EPAD // 128, 128)
    w_p = jnp.concatenate([edge_w, jnp.zeros((pad,), _f32)]).reshape(EPAD // 128, 128)

    f_in = jnp.pad(features, ((0, NPAD - NNODE), (0, 0)))

    nblk = NPAD // _BLK
    z1, p1b = pl.pallas_call(
        _ka_body,
        grid=(nblk,),
        in_specs=[
            pl.BlockSpec((_BLK, 100), lambda i: (i, 0)),
            pl.BlockSpec((100, 80), lambda i: (0, 0)),
            pl.BlockSpec((100, 80), lambda i: (0, 0)),
            pl.BlockSpec((80,), lambda i: (0,)),
        ],
        out_specs=[
            pl.BlockSpec((5, _BLK, 16), lambda i: (0, i, 0)),
            pl.BlockSpec((_BLK, 80), lambda i: (i, 0)),
        ],
        out_shape=[
            jax.ShapeDtypeStruct((5, NPAD, 16), _f32),
            jax.ShapeDtypeStruct((NPAD, 80), _f32),
        ],
    )(f_in, lin1_w.T, int1_w.T, lin1_b + int1_b)

    s1 = _spmm5(z1, src_p, dst_p, w_p)

    w2T = jnp.pad(lin2_w.T, ((0, 0), (0, 14)))
    wi2T = jnp.pad(int2_w.T, ((0, 0), (0, 14)))
    b2 = jnp.pad(lin2_b + int2_b, (0, 14))
    z2, p2b, f1 = pl.pallas_call(
        _kc_body,
        grid=(nblk,),
        in_specs=[
            pl.BlockSpec((5, _BLK, 16), lambda i: (0, i, 0)),
            pl.BlockSpec((_BLK, 80), lambda i: (i, 0)),
            pl.BlockSpec((80, 64), lambda i: (0, 0)),
            pl.BlockSpec((80, 64), lambda i: (0, 0)),
            pl.BlockSpec((64,), lambda i: (0,)),
        ],
        out_specs=[
            pl.BlockSpec((4, _BLK, 16), lambda i: (0, i, 0)),
            pl.BlockSpec((_BLK, 64), lambda i: (i, 0)),
            pl.BlockSpec((_BLK, 80), lambda i: (i, 0)),
        ],
        out_shape=[
            jax.ShapeDtypeStruct((4, NPAD, 16), _f32),
            jax.ShapeDtypeStruct((NPAD, 64), _f32),
            jax.ShapeDtypeStruct((NPAD, 80), _f32),
        ],
    )(s1, p1b, w2T, wi2T, b2)

    s2 = _spmm4(z2, src_p, dst_p, w_p)

    f2 = pl.pallas_call(
        _ke_body,
        grid=(nblk,),
        in_specs=[
            pl.BlockSpec((4, _BLK, 16), lambda i: (0, i, 0)),
            pl.BlockSpec((_BLK, 64), lambda i: (i, 0)),
        ],
        out_specs=pl.BlockSpec((_BLK, 64), lambda i: (i, 0)),
        out_shape=jax.ShapeDtypeStruct((NPAD, 64), _f32),
    )(s2, p2b)

    features_p = jnp.pad(features, ((0, NPAD - NNODE), (0, 12)))
    cat_idx = jnp.concatenate([userIdx, itemIdx + USER_NUM]).reshape(
        NCORE * NSUB, 2, 128)
    gf, g1, g2 = _gatherk(features_p, f1, f2, cat_idx)

    W = t1_w.T
    wfu = jnp.pad(W[0:100], ((0, 12), (0, 0)))
    w1u = W[100:180]
    w2u = jnp.pad(W[180:230], ((0, 14), (0, 0)))
    wfi = jnp.pad(W[230:330], ((0, 12), (0, 0)))
    w1i = W[330:410]
    w2i = jnp.pad(W[410:460], ((0, 14), (0, 0)))
    gf3 = gf.reshape(2, BATCH, 112)
    g13 = g1.reshape(2, BATCH, 80)
    g23 = g2.reshape(2, BATCH, 64)

    out = pl.pallas_call(
        _kg_body,
        grid=(BATCH // _BLKB,),
        in_specs=[
            pl.BlockSpec((1, _BLKB, 112), lambda i: (0, i, 0)),
            pl.BlockSpec((1, _BLKB, 112), lambda i: (1, i, 0)),
            pl.BlockSpec((1, _BLKB, 80), lambda i: (0, i, 0)),
            pl.BlockSpec((1, _BLKB, 80), lambda i: (1, i, 0)),
            pl.BlockSpec((1, _BLKB, 64), lambda i: (0, i, 0)),
            pl.BlockSpec((1, _BLKB, 64), lambda i: (1, i, 0)),
            pl.BlockSpec((112, 64), lambda i: (0, 0)),
            pl.BlockSpec((112, 64), lambda i: (0, 0)),
            pl.BlockSpec((80, 64), lambda i: (0, 0)),
            pl.BlockSpec((80, 64), lambda i: (0, 0)),
            pl.BlockSpec((64, 64), lambda i: (0, 0)),
            pl.BlockSpec((64, 64), lambda i: (0, 0)),
            pl.BlockSpec((64,), lambda i: (0,)),
            pl.BlockSpec((64, 32), lambda i: (0, 0)),
            pl.BlockSpec((32,), lambda i: (0,)),
            pl.BlockSpec((32, 1), lambda i: (0, 0)),
            pl.BlockSpec((1,), lambda i: (0,)),
        ],
        out_specs=pl.BlockSpec((_BLKB, 1), lambda i: (i, 0)),
        out_shape=jax.ShapeDtypeStruct((BATCH, 1), _f32),
    )(gf3, gf3, g13, g13, g23, g23,
      wfu, wfi, w1u, w1i, w2u, w2i, t1_b, t2_w.T, t2_b, t3_w.T, t3_b)
    return out.reshape(-1)


# speedup vs baseline: 11.0674x; 1.0222x over previous
import functools
import jax
import jax.numpy as jnp
from jax import lax
from jax.experimental import pallas as pl
from jax.experimental.pallas import tpu as pltpu
from jax.experimental.pallas import tpu_sc as plsc

USER_NUM = 10000
ITEM_NUM = 40000
NNODE = USER_NUM + ITEM_NUM
NPAD = 50176
NEDGE = 800000
NCORE = 2
NSUB = 16
EPAD = 819200
EDGES_PER_SUB = EPAD // NSUB
TILE = 2560
G = TILE // 128
NTILE = EDGES_PER_SUB // TILE
ROWS_PER_SUB = NPAD // NSUB
BATCH = 4096

_f32 = jnp.float32


def _bcast16(vec, r):
    idx = jax.lax.broadcast(jnp.int32(r), (16,))
    return jax.lax.gather(
        vec, idx[:, None],
        jax.lax.GatherDimensionNumbers(
            offset_dims=(), collapsed_slice_dims=(0,), start_index_map=(0,)),
        (1,), mode=jax.lax.GatherScatterMode.PROMISE_IN_BOUNDS)


def _make_spmm(nchunk):
    cpc = (nchunk + 1) // 2
    mesh = plsc.VectorSubcoreMesh(core_axis_name="c", subcore_axis_name="s")

    @functools.partial(
        pl.kernel, mesh=mesh,
        compiler_params=pltpu.CompilerParams(use_tc_tiling_on_sc=False),
        out_type=jax.ShapeDtypeStruct((nchunk, NPAD, 16), _f32),
        scratch_types=[
            pltpu.VMEM((G, 128), jnp.int32),
            pltpu.VMEM((G, 128), jnp.int32),
            pltpu.VMEM((G, 128), _f32),
            pltpu.VMEM((TILE, 16), _f32),
            pltpu.VMEM_SHARED((NPAD, 16), _f32),
            pltpu.SemaphoreType.DMA,
            pltpu.SemaphoreType.DMA,
            pltpu.SemaphoreType.DMA,
        ],
    )
    def spmm(z_hbm, src_hbm, dst_hbm, w_hbm, out_hbm,
             src_v, dst_v, w_v, rows_v, acc_sh, sem, semi, sems):
        c = lax.axis_index("c")
        s = lax.axis_index("s")
        row0 = s * ROWS_PER_SUB

        for t in range(cpc):
            j = t * 2 + c

            @pl.when(j < nchunk)
            def _chunk():
                def zb(i, x):
                    rows_v[i] = jnp.zeros((16,), _f32)
                    return x
                lax.fori_loop(0, TILE, zb, 0)
                pltpu.sync_copy(rows_v.at[pl.ds(0, 2048)],
                                acc_sh.at[pl.ds(row0, 2048)])
                pltpu.sync_copy(rows_v.at[pl.ds(0, ROWS_PER_SUB - 2048)],
                                acc_sh.at[pl.ds(row0 + 2048, ROWS_PER_SUB - 2048)])
                plsc.subcore_barrier()

                def tile_body(ti, x):
                    roff = s * (EDGES_PER_SUB // 128) + ti * G
                    d1 = pltpu.async_copy(src_hbm.at[pl.ds(roff, G)], src_v, semi)
                    d2 = pltpu.async_copy(dst_hbm.at[pl.ds(roff, G)], dst_v, semi)
                    d3 = pltpu.async_copy(w_hbm.at[pl.ds(roff, G)], w_v, semi)
                    d1.wait(); d2.wait(); d3.wait()
                    gds = [pltpu.async_copy(
                               z_hbm.at[j].at[src_v.at[g]],
                               rows_v.at[pl.ds(g * 128, 128)], sem)
                           for g in range(G)]
                    for g in range(G):
                        gds[g].wait()
                        def mulbody(e16, y, g=g):
                            wv = w_v[g, pl.ds(e16 * 16, 16)]
                            base = g * 128 + e16 * 16
                            for r in range(16):
                                rows_v[base + r] = rows_v[base + r] * _bcast16(wv, r)
                            return y
                        lax.fori_loop(0, 8, mulbody, 0)
                        pltpu.async_copy(rows_v.at[pl.ds(g * 128, 128)],
                                         acc_sh.at[dst_v.at[g]], sems, add=True)
                    for g in range(G):
                        pltpu.make_async_copy(rows_v.at[pl.ds(g * 128, 128)],
                                              acc_sh.at[dst_v.at[g]], sems).wait()
                    return x
                lax.fori_loop(0, NTILE, tile_body, 0)
                plsc.subcore_barrier()
                pltpu.sync_copy(acc_sh.at[pl.ds(row0, ROWS_PER_SUB)],
                                out_hbm.at[j, pl.ds(row0, ROWS_PER_SUB)])
    return spmm


_spmm5 = _make_spmm(5)
_spmm4 = _make_spmm(4)


_BLK = 1024


def _ka_body(f_ref, w1_ref, wi_ref, b_ref, z_ref, pb_ref):
    f = f_ref[...]
    p = jnp.dot(f, w1_ref[...], preferred_element_type=_f32)
    q = jnp.dot(f * f, wi_ref[...], preferred_element_type=_f32)
    z = p + q
    z_ref[...] = jnp.transpose(z.reshape(_BLK, 5, 16), (1, 0, 2))
    pb_ref[...] = p + b_ref[...]


def _kc_body(s_ref, pb_ref, w2_ref, wi2_ref, b2_ref, z2_ref, p2b_ref, f1_ref):
    sflat = jnp.transpose(s_ref[...], (1, 0, 2)).reshape(_BLK, 80)
    f1 = jax.nn.relu(sflat + pb_ref[...])
    f1_ref[...] = f1
    p = jnp.dot(f1, w2_ref[...], preferred_element_type=_f32)
    q = jnp.dot(f1 * f1, wi2_ref[...], preferred_element_type=_f32)
    z = p + q
    z2_ref[...] = jnp.transpose(z.reshape(_BLK, 4, 16), (1, 0, 2))
    p2b_ref[...] = p + b2_ref[...]


def _ke_body(s_ref, p2b_ref, f2_ref):
    sflat = jnp.transpose(s_ref[...], (1, 0, 2)).reshape(_BLK, 64)
    f2_ref[...] = jax.nn.relu(sflat + p2b_ref[...])


def _make_gatherk():
    mesh = plsc.VectorSubcoreMesh(core_axis_name="c", subcore_axis_name="s")
    nw = NCORE * NSUB
    bpw = 2 * BATCH // nw

    @functools.partial(
        pl.kernel, mesh=mesh,
        compiler_params=pltpu.CompilerParams(use_tc_tiling_on_sc=False),
        out_type=[jax.ShapeDtypeStruct((2 * BATCH, 112), _f32),
                  jax.ShapeDtypeStruct((2 * BATCH, 80), _f32),
                  jax.ShapeDtypeStruct((2 * BATCH, 64), _f32)],
        scratch_types=[
            pltpu.VMEM((2, 128), jnp.int32),
            pltpu.VMEM((bpw, 112), _f32),
            pltpu.VMEM((bpw, 80), _f32),
            pltpu.VMEM((bpw, 64), _f32),
            pltpu.SemaphoreType.DMA,
        ],
    )
    def gk(fp_hbm, f1_hbm, f2_hbm, idx_hbm, gf_hbm, g1_hbm, g2_hbm,
           idx_v, bf, b1, b2, sem):
        c = lax.axis_index("c")
        s = lax.axis_index("s")
        wid = s * NCORE + c
        base = wid * bpw
        pltpu.sync_copy(idx_hbm.at[wid], idx_v)
        for g in range(2):
            pltpu.async_copy(fp_hbm.at[idx_v.at[g]],
                             bf.at[pl.ds(g * 128, 128)], sem).wait()
            pltpu.async_copy(f1_hbm.at[idx_v.at[g]],
                             b1.at[pl.ds(g * 128, 128)], sem).wait()
            pltpu.async_copy(f2_hbm.at[idx_v.at[g]],
                             b2.at[pl.ds(g * 128, 128)], sem).wait()
        pltpu.sync_copy(bf, gf_hbm.at[pl.ds(base, bpw)])
        pltpu.sync_copy(b1, g1_hbm.at[pl.ds(base, bpw)])
        pltpu.sync_copy(b2, g2_hbm.at[pl.ds(base, bpw)])
    return gk


_gatherk = _make_gatherk()

_BLKB = 512


def _kg_body(gfu, gfi, g1u, g1i, g2u, g2i,
             wfu, wfi, w1u, w1i, w2u, w2i, b1, w2, b2, w3, b3, o_ref):
    h = (jnp.dot(gfu[0], wfu[...], preferred_element_type=_f32)
         + jnp.dot(gfi[0], wfi[...], preferred_element_type=_f32)
         + jnp.dot(g1u[0], w1u[...], preferred_element_type=_f32)
         + jnp.dot(g1i[0], w1i[...], preferred_element_type=_f32)
         + jnp.dot(g2u[0], w2u[...], preferred_element_type=_f32)
         + jnp.dot(g2i[0], w2i[...], preferred_element_type=_f32)) + b1[...]
    h = jax.nn.relu(h)
    h = jax.nn.relu(jnp.dot(h, w2[...], preferred_element_type=_f32) + b2[...])
    o_ref[...] = jnp.dot(h, w3[...], preferred_element_type=_f32) + b3[...]


def kernel(userIdx, itemIdx, edge_src, edge_dst, edge_w, uEmbd, iEmbd,
           lin1_w, lin1_b, int1_w, int1_b, lin2_w, lin2_b, int2_w, int2_b,
           t1_w, t1_b, t2_w, t2_b, t3_w, t3_b):
    features = jnp.concatenate([uEmbd, iEmbd], axis=0)

    pad = EPAD - NEDGE
    padidx = (jnp.arange(pad, dtype=jnp.int32) * 61) % NNODE
    src_p = jnp.concatenate([edge_src, padidx]).reshape(EPAD // 128, 128)
    dst_p = jnp.concatenate([edge_dst, padidx]).reshape(EPAD // 128, 128)
    w_p = jnp.concatenate([edge_w, jnp.zeros((pad,), _f32)]).reshape(EPAD // 128, 128)

    f_in = jnp.pad(features, ((0, NPAD - NNODE), (0, 0)))

    nblk = NPAD // _BLK
    z1, p1b = pl.pallas_call(
        _ka_body,
        grid=(nblk,),
        in_specs=[
            pl.BlockSpec((_BLK, 100), lambda i: (i, 0)),
            pl.BlockSpec((100, 80), lambda i: (0, 0)),
            pl.BlockSpec((100, 80), lambda i: (0, 0)),
            pl.BlockSpec((80,), lambda i: (0,)),
        ],
        out_specs=[
            pl.BlockSpec((5, _BLK, 16), lambda i: (0, i, 0)),
            pl.BlockSpec((_BLK, 80), lambda i: (i, 0)),
        ],
        out_shape=[
            jax.ShapeDtypeStruct((5, NPAD, 16), _f32),
            jax.ShapeDtypeStruct((NPAD, 80), _f32),
        ],
    )(f_in, lin1_w.T, int1_w.T, lin1_b + int1_b)

    s1 = _spmm5(z1, src_p, dst_p, w_p)

    w2T = jnp.pad(lin2_w.T, ((0, 0), (0, 14)))
    wi2T = jnp.pad(int2_w.T, ((0, 0), (0, 14)))
    b2 = jnp.pad(lin2_b + int2_b, (0, 14))
    z2, p2b, f1 = pl.pallas_call(
        _kc_body,
        grid=(nblk,),
        in_specs=[
            pl.BlockSpec((5, _BLK, 16), lambda i: (0, i, 0)),
            pl.BlockSpec((_BLK, 80), lambda i: (i, 0)),
            pl.BlockSpec((80, 64), lambda i: (0, 0)),
            pl.BlockSpec((80, 64), lambda i: (0, 0)),
            pl.BlockSpec((64,), lambda i: (0,)),
        ],
        out_specs=[
            pl.BlockSpec((4, _BLK, 16), lambda i: (0, i, 0)),
            pl.BlockSpec((_BLK, 64), lambda i: (i, 0)),
            pl.BlockSpec((_BLK, 80), lambda i: (i, 0)),
        ],
        out_shape=[
            jax.ShapeDtypeStruct((4, NPAD, 16), _f32),
            jax.ShapeDtypeStruct((NPAD, 64), _f32),
            jax.ShapeDtypeStruct((NPAD, 80), _f32),
        ],
    )(s1, p1b, w2T, wi2T, b2)

    s2 = _spmm4(z2, src_p, dst_p, w_p)

    f2 = pl.pallas_call(
        _ke_body,
        grid=(nblk,),
        in_specs=[
            pl.BlockSpec((4, _BLK, 16), lambda i: (0, i, 0)),
            pl.BlockSpec((_BLK, 64), lambda i: (i, 0)),
        ],
        out_specs=pl.BlockSpec((_BLK, 64), lambda i: (i, 0)),
        out_shape=jax.ShapeDtypeStruct((NPAD, 64), _f32),
    )(s2, p2b)

    features_p = jnp.pad(features, ((0, NPAD - NNODE), (0, 12)))
    cat_idx = jnp.concatenate([userIdx, itemIdx + USER_NUM]).reshape(
        NCORE * NSUB, 2, 128)
    gf, g1, g2 = _gatherk(features_p, f1, f2, cat_idx)

    W = t1_w.T
    wfu = jnp.pad(W[0:100], ((0, 12), (0, 0)))
    w1u = W[100:180]
    w2u = jnp.pad(W[180:230], ((0, 14), (0, 0)))
    wfi = jnp.pad(W[230:330], ((0, 12), (0, 0)))
    w1i = W[330:410]
    w2i = jnp.pad(W[410:460], ((0, 14), (0, 0)))
    gf3 = gf.reshape(2, BATCH, 112)
    g13 = g1.reshape(2, BATCH, 80)
    g23 = g2.reshape(2, BATCH, 64)

    out = pl.pallas_call(
        _kg_body,
        grid=(BATCH // _BLKB,),
        in_specs=[
            pl.BlockSpec((1, _BLKB, 112), lambda i: (0, i, 0)),
            pl.BlockSpec((1, _BLKB, 112), lambda i: (1, i, 0)),
            pl.BlockSpec((1, _BLKB, 80), lambda i: (0, i, 0)),
            pl.BlockSpec((1, _BLKB, 80), lambda i: (1, i, 0)),
            pl.BlockSpec((1, _BLKB, 64), lambda i: (0, i, 0)),
            pl.BlockSpec((1, _BLKB, 64), lambda i: (1, i, 0)),
            pl.BlockSpec((112, 64), lambda i: (0, 0)),
            pl.BlockSpec((112, 64), lambda i: (0, 0)),
            pl.BlockSpec((80, 64), lambda i: (0, 0)),
            pl.BlockSpec((80, 64), lambda i: (0, 0)),
            pl.BlockSpec((64, 64), lambda i: (0, 0)),
            pl.BlockSpec((64, 64), lambda i: (0, 0)),
            pl.BlockSpec((64,), lambda i: (0,)),
            pl.BlockSpec((64, 32), lambda i: (0, 0)),
            pl.BlockSpec((32,), lambda i: (0,)),
            pl.BlockSpec((32, 1), lambda i: (0, 0)),
            pl.BlockSpec((1,), lambda i: (0,)),
        ],
        out_specs=pl.BlockSpec((_BLKB, 1), lambda i: (i, 0)),
        out_shape=jax.ShapeDtypeStruct((BATCH, 1), _f32),
    )(gf3, gf3, g13, g13, g23, g23,
      wfu, wfi, w1u, w1i, w2u, w2i, t1_b, t2_w.T, t2_b, t3_w.T, t3_b)
    return out.reshape(-1)
